# Initial kernel scaffold; baseline (speedup 1.0000x reference)
import jax
import jax.numpy as jnp
from jax.experimental import pallas as pl


def kernel(x_nchw, stem_w, stem_b, s0_b0_conv1_w, s0_b0_conv1_b, s0_b0_conv2_w, s0_b0_conv2_b, s0_b0_conv3_w, s0_b0_conv3_b, s0_b0_downsample_w, s0_b0_downsample_b, s0_b1_conv1_w, s0_b1_conv1_b, s0_b1_conv2_w, s0_b1_conv2_b, s0_b1_conv3_w, s0_b1_conv3_b, s0_b2_conv1_w, s0_b2_conv1_b, s0_b2_conv2_w, s0_b2_conv2_b, s0_b2_conv3_w, s0_b2_conv3_b, s1_b0_conv1_w, s1_b0_conv1_b, s1_b0_conv2_w, s1_b0_conv2_b, s1_b0_conv3_w, s1_b0_conv3_b, s1_b0_downsample_w, s1_b0_downsample_b, s1_b1_conv1_w, s1_b1_conv1_b, s1_b1_conv2_w, s1_b1_conv2_b, s1_b1_conv3_w, s1_b1_conv3_b, s1_b2_conv1_w, s1_b2_conv1_b, s1_b2_conv2_w, s1_b2_conv2_b, s1_b2_conv3_w, s1_b2_conv3_b, s1_b3_conv1_w, s1_b3_conv1_b, s1_b3_conv2_w, s1_b3_conv2_b, s1_b3_conv3_w, s1_b3_conv3_b, s2_b0_conv1_w, s2_b0_conv1_b, s2_b0_conv2_w, s2_b0_conv2_b, s2_b0_conv3_w, s2_b0_conv3_b, s2_b0_downsample_w, s2_b0_downsample_b, s2_b1_conv1_w, s2_b1_conv1_b, s2_b1_conv2_w, s2_b1_conv2_b, s2_b1_conv3_w, s2_b1_conv3_b, s2_b2_conv1_w, s2_b2_conv1_b, s2_b2_conv2_w, s2_b2_conv2_b, s2_b2_conv3_w, s2_b2_conv3_b, s2_b3_conv1_w, s2_b3_conv1_b, s2_b3_conv2_w, s2_b3_conv2_b, s2_b3_conv3_w, s2_b3_conv3_b, s2_b4_conv1_w, s2_b4_conv1_b, s2_b4_conv2_w, s2_b4_conv2_b, s2_b4_conv3_w, s2_b4_conv3_b, s2_b5_conv1_w, s2_b5_conv1_b, s2_b5_conv2_w, s2_b5_conv2_b, s2_b5_conv3_w, s2_b5_conv3_b, s3_b0_conv1_w, s3_b0_conv1_b, s3_b0_conv2_w, s3_b0_conv2_b, s3_b0_conv3_w, s3_b0_conv3_b, s3_b0_downsample_w, s3_b0_downsample_b, s3_b1_conv1_w, s3_b1_conv1_b, s3_b1_conv2_w, s3_b1_conv2_b, s3_b1_conv3_w, s3_b1_conv3_b, s3_b2_conv1_w, s3_b2_conv1_b, s3_b2_conv2_w, s3_b2_conv2_b, s3_b2_conv3_w, s3_b2_conv3_b, fc_w, fc_b):
    raise NotImplementedError("write your pallas kernel here")


# speedup vs baseline: 1.3899x; 1.3899x over previous
import functools

import jax
import jax.numpy as jnp
from jax.experimental import pallas as pl
from jax.experimental.pallas import tpu as pltpu


def _ru(x, m):
    return ((x + m - 1) // m) * m


def _pick_tm(m):
    for t in (256, 224, 192, 160, 128, 112, 96, 80, 64, 48, 32, 16):
        if m % t == 0:
            return t
    return 0


def _gemm_body(relu, has_res):
    if has_res:
        def body(a_ref, w_ref, b_ref, r_ref, o_ref):
            y = jnp.dot(a_ref[...], w_ref[...],
                        preferred_element_type=jnp.float32)
            y = y + b_ref[...] + r_ref[...].astype(jnp.float32)
            if relu:
                y = jnp.maximum(y, 0.0)
            o_ref[...] = y.astype(o_ref.dtype)
    else:
        def body(a_ref, w_ref, b_ref, o_ref):
            y = jnp.dot(a_ref[...], w_ref[...],
                        preferred_element_type=jnp.float32)
            y = y + b_ref[...]
            if relu:
                y = jnp.maximum(y, 0.0)
            o_ref[...] = y.astype(o_ref.dtype)
    return body


@functools.lru_cache(maxsize=None)
def _gemm_call(Mp, K, Np, tm, tn, relu, has_res, out_dtype):
    grid = (Mp // tm, Np // tn)
    in_specs = [
        pl.BlockSpec((tm, K), lambda i, j: (i, 0)),
        pl.BlockSpec((K, tn), lambda i, j: (0, j)),
        pl.BlockSpec((1, tn), lambda i, j: (0, j)),
    ]
    if has_res:
        in_specs.append(pl.BlockSpec((tm, tn), lambda i, j: (i, j)))
    return pl.pallas_call(
        _gemm_body(relu, has_res),
        out_shape=jax.ShapeDtypeStruct((Mp, Np), out_dtype),
        grid=grid,
        in_specs=in_specs,
        out_specs=pl.BlockSpec((tm, tn), lambda i, j: (i, j)),
        compiler_params=pltpu.CompilerParams(
            dimension_semantics=("parallel", "parallel")),
    )


def _gemm(a, w, b, relu=False, residual=None, out_dtype=jnp.bfloat16):
    M, K = a.shape
    Kw, Np = w.shape
    assert Kw == K, (Kw, K)
    N = b.shape[0]
    tm = _pick_tm(M)
    Mp = M
    if tm == 0:
        tm = 256
        Mp = _ru(M, 256)
        a = jnp.pad(a, ((0, Mp - M), (0, 0)))
    tn = min(Np, 256)
    bp = b.astype(jnp.float32).reshape(1, N)
    if Np != N:
        bp = jnp.pad(bp, ((0, 0), (0, Np - N)))
    args = [a, w, bp]
    if residual is not None:
        r = residual
        assert r.shape[1] == Np
        if Mp != M:
            r = jnp.pad(r, ((0, Mp - M), (0, 0)))
        args.append(r)
    fn = _gemm_call(Mp, K, Np, tm, tn, bool(relu), residual is not None,
                    out_dtype)
    return fn(*args)[:M]


def _taps_body(offs, cin, tm, relu):
    def body(a_ref, w_ref, b_ref, o_ref):
        i = pl.program_id(1)
        m0 = i * tm
        acc = None
        for t, (off_row, off_col) in enumerate(offs):
            base = pl.multiple_of(m0 + off_row, 16)
            raw = a_ref[0, pl.ds(base, tm + 16), :]
            a = raw[off_col:off_col + tm, :]
            p = jnp.dot(a, w_ref[t * cin:(t + 1) * cin, :],
                        preferred_element_type=jnp.float32)
            acc = p if acc is None else acc + p
        y = acc + b_ref[...]
        if relu:
            y = jnp.maximum(y, 0.0)
        o_ref[0] = y.astype(o_ref.dtype)
    return body


@functools.lru_cache(maxsize=None)
def _taps_call(B, L, cin, offs, Np, Mp, tm, tn, relu):
    Kw = len(offs) * cin
    grid = (B, Mp // tm, Np // tn)
    return pl.pallas_call(
        _taps_body(offs, cin, tm, relu),
        out_shape=jax.ShapeDtypeStruct((B, Mp, Np), jnp.bfloat16),
        grid=grid,
        in_specs=[
            pl.BlockSpec((1, L, cin), lambda b, i, j: (b, 0, 0)),
            pl.BlockSpec((Kw, tn), lambda b, i, j: (0, j)),
            pl.BlockSpec((1, tn), lambda b, i, j: (0, j)),
        ],
        out_specs=pl.BlockSpec((1, tm, tn), lambda b, i, j: (b, i, j)),
        compiler_params=pltpu.CompilerParams(
            dimension_semantics=("parallel", "parallel", "parallel")),
    )


def _pad_bias(b, Np):
    N = b.shape[0]
    bp = b.astype(jnp.float32).reshape(1, N)
    if Np != N:
        bp = jnp.pad(bp, ((0, 0), (0, Np - N)))
    return bp


def _conv3x3_s1(x, w, b, relu=True):
    B, H, W, C = x.shape
    N = b.shape[0]
    Np = w.shape[1]
    wp = _ru(W + 2, 16)
    M_img = H * wp
    tm = _pick_tm(M_img)
    L = _ru(M_img + 2 * wp + 16, 16)
    xp = jnp.pad(x, ((0, 0), (1, 1), (1, wp - W - 1), (0, 0)))
    xp = xp.reshape(B, (H + 2) * wp, C)
    xp = jnp.pad(xp, ((0, 0), (0, L - (H + 2) * wp), (0, 0)))
    offs = tuple((di * wp, dj) for di in range(3) for dj in range(3))
    fn = _taps_call(B, L, C, offs, Np, M_img, tm, Np, bool(relu))
    out = fn(xp, w[:9 * C], _pad_bias(b, Np))
    return out[:, :, :N].reshape(B, H, wp, N)[:, :, :W, :]


def _conv3x3_s2(x, w, b, relu=True):
    B, H, W, C = x.shape
    H2, W2 = H // 2, W // 2
    N = b.shape[0]
    Np = w.shape[1]
    x2 = x.reshape(B, H2, 2, W2, 2, C).transpose(0, 1, 3, 2, 4, 5)
    x2 = x2.reshape(B, H2, W2, 4 * C)
    w9 = w[:9 * C].reshape(3, 3, C, Np)
    w16 = jnp.zeros((4, 4, C, Np), w.dtype).at[1:, 1:].set(w9)
    wk = w16.reshape(2, 2, 2, 2, C, Np).transpose(0, 2, 1, 3, 4, 5)
    wk = wk.reshape(16 * C, Np)
    wp = _ru(W2 + 2, 16)
    M_img = H2 * wp
    tm = _pick_tm(M_img)
    L = (H2 + 2) * wp
    xp = jnp.pad(x2, ((0, 0), (1, 1), (1, wp - W2 - 1), (0, 0)))
    xp = xp.reshape(B, L, 4 * C)
    offs = ((0, 0), (0, 1), (wp, 0), (wp, 1))
    fn = _taps_call(B, L, 4 * C, offs, Np, M_img, tm, Np, bool(relu))
    out = fn(xp, wk, _pad_bias(b, Np))
    return out[:, :, :N].reshape(B, H2, wp, N)[:, :, :W2, :]


def _conv1x1(x, w, b, relu=False, residual=None, stride=1):
    B, H, W, C = x.shape
    if stride != 1:
        x = x[:, ::stride, ::stride, :]
        H, W = x.shape[1], x.shape[2]
    a = x.reshape(B * H * W, C)
    out = _gemm(a, w, b, relu=relu, residual=residual)
    return out, (B, H, W)


def _stem(x_nchw, w, b):
    x = jnp.transpose(x_nchw, (0, 2, 3, 1)).astype(jnp.bfloat16)
    B, H, W, _ = x.shape
    H2, W2 = H // 2, W // 2
    Np = w.shape[1]
    x2 = x.reshape(B, H2, 2, W2, 2, 3).transpose(0, 1, 3, 2, 4, 5)
    x2 = x2.reshape(B, H2, W2, 12)
    P = jnp.pad(x2, ((0, 0), (2, 1), (2, 3), (0, 0)))
    S = jnp.concatenate(
        [P[:, I:I + H2, J:J + W2, :] for I in range(4) for J in range(4)],
        axis=-1)
    a = S.reshape(B * H2 * W2, 192)
    w7 = w[:147].reshape(7, 7, 3, Np)
    w8 = jnp.zeros((8, 8, 3, Np), w.dtype).at[1:, 1:].set(w7)
    wk = w8.reshape(4, 2, 4, 2, 3, Np).transpose(0, 2, 1, 3, 4, 5)
    wk = wk.reshape(192, Np)
    out = _gemm(a, wk, b, relu=True)
    return out[:, :b.shape[0]].reshape(B, H2, W2, b.shape[0])


def _maxpool_3x3_s2(x):
    B, H, W, C = x.shape
    xp = jnp.pad(x, ((0, 0), (1, 1), (1, 1), (0, 0)),
                 constant_values=-jnp.inf)
    Ho = (H - 1) // 2 + 1
    Wo = (W - 1) // 2 + 1
    out = None
    for i in range(3):
        for j in range(3):
            s = xp[:, i:i + 2 * Ho - 1:2, j:j + 2 * Wo - 1:2, :]
            out = s if out is None else jnp.maximum(out, s)
    return out


def _head_body(a_ref, w_ref, b_ref, o_ref):
    hw = a_ref.shape[1]
    xm = jnp.sum(a_ref[...].astype(jnp.float32), axis=1) * (1.0 / hw)
    o_ref[...] = jnp.dot(xm.astype(jnp.bfloat16), w_ref[...],
                         preferred_element_type=jnp.float32) + b_ref[...]


@functools.lru_cache(maxsize=None)
def _head_call(B, HW, K, Np, tn):
    grid = (Np // tn,)
    return pl.pallas_call(
        _head_body,
        out_shape=jax.ShapeDtypeStruct((B, Np), jnp.float32),
        grid=grid,
        in_specs=[
            pl.BlockSpec((B, HW, K), lambda j: (0, 0, 0)),
            pl.BlockSpec((K, tn), lambda j: (0, j)),
            pl.BlockSpec((1, tn), lambda j: (0, j)),
        ],
        out_specs=pl.BlockSpec((B, tn), lambda j: (0, j)),
        compiler_params=pltpu.CompilerParams(
            dimension_semantics=("parallel",)),
    )


def _head(x, w, b):
    B, H, W, C = x.shape
    K, Np = w.shape
    assert K == C
    N = b.shape[0]
    xr = x.reshape(B, H * W, C)
    fn = _head_call(B, H * W, C, Np, min(Np, 512))
    return fn(xr, w, _pad_bias(b, Np))[:, :N]


def _bottleneck(x, blk, stride):
    c1w, c1b, c2w, c2b, c3w, c3b = blk[:6]
    out, _ = _conv1x1(x, c1w, c1b, relu=True)
    B, H, W, _ = x.shape
    out = out[:, :c1b.shape[0]].reshape(B, H, W, c1b.shape[0])
    if stride == 1:
        out = _conv3x3_s1(out, c2w, c2b, relu=True)
    else:
        out = _conv3x3_s2(out, c2w, c2b, relu=True)
    if len(blk) > 6:
        idn, _ = _conv1x1(x, blk[6], blk[7], relu=False, stride=stride)
    else:
        Ho, Wo = out.shape[1], out.shape[2]
        idn = x.reshape(B * Ho * Wo, -1)
    Bo, Ho, Wo, Co = out.shape
    res, _ = _conv1x1(out, c3w, c3b, relu=True, residual=idn)
    N3 = c3b.shape[0]
    return res[:, :N3].reshape(Bo, Ho, Wo, N3)


def _forward(x_nchw, stem_w, stem_b, blocks, fc_w, fc_b):
    x = _stem(x_nchw, stem_w, stem_b)
    x = _maxpool_3x3_s2(x)
    cfg = [3, 4, 6, 3]
    bi = 0
    for si, nb in enumerate(cfg):
        for k in range(nb):
            stride = 2 if (si > 0 and k == 0) else 1
            x = _bottleneck(x, blocks[bi], stride)
            bi += 1
    return _head(x, fc_w, fc_b)


_forward_jit = jax.jit(_forward, static_argnames=())


def kernel(x_nchw, stem_w, stem_b, s0_b0_conv1_w, s0_b0_conv1_b, s0_b0_conv2_w, s0_b0_conv2_b, s0_b0_conv3_w, s0_b0_conv3_b, s0_b0_downsample_w, s0_b0_downsample_b, s0_b1_conv1_w, s0_b1_conv1_b, s0_b1_conv2_w, s0_b1_conv2_b, s0_b1_conv3_w, s0_b1_conv3_b, s0_b2_conv1_w, s0_b2_conv1_b, s0_b2_conv2_w, s0_b2_conv2_b, s0_b2_conv3_w, s0_b2_conv3_b, s1_b0_conv1_w, s1_b0_conv1_b, s1_b0_conv2_w, s1_b0_conv2_b, s1_b0_conv3_w, s1_b0_conv3_b, s1_b0_downsample_w, s1_b0_downsample_b, s1_b1_conv1_w, s1_b1_conv1_b, s1_b1_conv2_w, s1_b1_conv2_b, s1_b1_conv3_w, s1_b1_conv3_b, s1_b2_conv1_w, s1_b2_conv1_b, s1_b2_conv2_w, s1_b2_conv2_b, s1_b2_conv3_w, s1_b2_conv3_b, s1_b3_conv1_w, s1_b3_conv1_b, s1_b3_conv2_w, s1_b3_conv2_b, s1_b3_conv3_w, s1_b3_conv3_b, s2_b0_conv1_w, s2_b0_conv1_b, s2_b0_conv2_w, s2_b0_conv2_b, s2_b0_conv3_w, s2_b0_conv3_b, s2_b0_downsample_w, s2_b0_downsample_b, s2_b1_conv1_w, s2_b1_conv1_b, s2_b1_conv2_w, s2_b1_conv2_b, s2_b1_conv3_w, s2_b1_conv3_b, s2_b2_conv1_w, s2_b2_conv1_b, s2_b2_conv2_w, s2_b2_conv2_b, s2_b2_conv3_w, s2_b2_conv3_b, s2_b3_conv1_w, s2_b3_conv1_b, s2_b3_conv2_w, s2_b3_conv2_b, s2_b3_conv3_w, s2_b3_conv3_b, s2_b4_conv1_w, s2_b4_conv1_b, s2_b4_conv2_w, s2_b4_conv2_b, s2_b4_conv3_w, s2_b4_conv3_b, s2_b5_conv1_w, s2_b5_conv1_b, s2_b5_conv2_w, s2_b5_conv2_b, s2_b5_conv3_w, s2_b5_conv3_b, s3_b0_conv1_w, s3_b0_conv1_b, s3_b0_conv2_w, s3_b0_conv2_b, s3_b0_conv3_w, s3_b0_conv3_b, s3_b0_downsample_w, s3_b0_downsample_b, s3_b1_conv1_w, s3_b1_conv1_b, s3_b1_conv2_w, s3_b1_conv2_b, s3_b1_conv3_w, s3_b1_conv3_b, s3_b2_conv1_w, s3_b2_conv1_b, s3_b2_conv2_w, s3_b2_conv2_b, s3_b2_conv3_w, s3_b2_conv3_b, fc_w, fc_b):
    A = dict(locals())
    cfg = [3, 4, 6, 3]
    blocks = []
    for si, nb in enumerate(cfg):
        for b in range(nb):
            names = [f's{si}_b{b}_{n}_{t}'
                     for n in ('conv1', 'conv2', 'conv3') for t in ('w', 'b')]
            blk = [A[n] for n in names]
            dwn = f's{si}_b{b}_downsample_w'
            if dwn in A:
                blk += [A[dwn], A[f's{si}_b{b}_downsample_b']]
            blocks.append(tuple(blk))
    return _forward_jit(x_nchw, stem_w, stem_b, blocks, fc_w, fc_b)


# speedup vs baseline: 1.7616x; 1.2674x over previous
import functools

import jax
import jax.numpy as jnp
from jax.experimental import pallas as pl
from jax.experimental.pallas import tpu as pltpu


def _ru(x, m):
    return ((x + m - 1) // m) * m


def _pick_tm(m):
    for t in (256, 224, 192, 160, 128, 112, 96, 80, 64, 48, 32, 16):
        if m % t == 0:
            return t
    return 0


def _gemm_body(relu, has_res):
    if has_res:
        def body(a_ref, w_ref, b_ref, r_ref, o_ref):
            y = jnp.dot(a_ref[...], w_ref[...],
                        preferred_element_type=jnp.float32)
            y = y + b_ref[...] + r_ref[...].astype(jnp.float32)
            if relu:
                y = jnp.maximum(y, 0.0)
            o_ref[...] = y.astype(o_ref.dtype)
    else:
        def body(a_ref, w_ref, b_ref, o_ref):
            y = jnp.dot(a_ref[...], w_ref[...],
                        preferred_element_type=jnp.float32)
            y = y + b_ref[...]
            if relu:
                y = jnp.maximum(y, 0.0)
            o_ref[...] = y.astype(o_ref.dtype)
    return body


@functools.lru_cache(maxsize=None)
def _gemm_call(Mp, K, Np, tm, tn, relu, has_res, out_dtype):
    grid = (Mp // tm, Np // tn)
    in_specs = [
        pl.BlockSpec((tm, K), lambda i, j: (i, 0)),
        pl.BlockSpec((K, tn), lambda i, j: (0, j)),
        pl.BlockSpec((1, tn), lambda i, j: (0, j)),
    ]
    if has_res:
        in_specs.append(pl.BlockSpec((tm, tn), lambda i, j: (i, j)))
    return pl.pallas_call(
        _gemm_body(relu, has_res),
        out_shape=jax.ShapeDtypeStruct((Mp, Np), out_dtype),
        grid=grid,
        in_specs=in_specs,
        out_specs=pl.BlockSpec((tm, tn), lambda i, j: (i, j)),
        compiler_params=pltpu.CompilerParams(
            dimension_semantics=("parallel", "parallel")),
    )


def _gemm(a, w, b, relu=False, residual=None, out_dtype=jnp.bfloat16):
    M, K = a.shape
    Kw, Np = w.shape
    assert Kw == K, (Kw, K)
    N = b.shape[0]
    tm = _pick_tm(M)
    Mp = M
    if tm == 0:
        tm = 256
        Mp = _ru(M, 256)
        a = jnp.pad(a, ((0, Mp - M), (0, 0)))
    tn = min(Np, 256)
    bp = b.astype(jnp.float32).reshape(1, N)
    if Np != N:
        bp = jnp.pad(bp, ((0, 0), (0, Np - N)))
    args = [a, w, bp]
    if residual is not None:
        r = residual
        assert r.shape[1] == Np
        if Mp != M:
            r = jnp.pad(r, ((0, Mp - M), (0, 0)))
        args.append(r)
    fn = _gemm_call(Mp, K, Np, tm, tn, bool(relu), residual is not None,
                    out_dtype)
    return fn(*args)[:M]


def _taps_body(offs, cin, tm, relu):
    def body(a_ref, w_ref, b_ref, o_ref):
        i = pl.program_id(1)
        m0 = i * tm
        acc = None
        for t, (off_row, off_col) in enumerate(offs):
            base = pl.multiple_of(m0 + off_row, 16)
            raw = a_ref[0, pl.ds(base, tm + 16), :]
            a = raw[off_col:off_col + tm, :]
            p = jnp.dot(a, w_ref[t * cin:(t + 1) * cin, :],
                        preferred_element_type=jnp.float32)
            acc = p if acc is None else acc + p
        y = acc + b_ref[...]
        if relu:
            y = jnp.maximum(y, 0.0)
        o_ref[0] = y.astype(o_ref.dtype)
    return body


@functools.lru_cache(maxsize=None)
def _taps_call(B, L, cin, offs, Np, Mp, tm, tn, relu):
    Kw = len(offs) * cin
    grid = (B, Mp // tm, Np // tn)
    return pl.pallas_call(
        _taps_body(offs, cin, tm, relu),
        out_shape=jax.ShapeDtypeStruct((B, Mp, Np), jnp.bfloat16),
        grid=grid,
        in_specs=[
            pl.BlockSpec((1, L, cin), lambda b, i, j: (b, 0, 0)),
            pl.BlockSpec((Kw, tn), lambda b, i, j: (0, j)),
            pl.BlockSpec((1, tn), lambda b, i, j: (0, j)),
        ],
        out_specs=pl.BlockSpec((1, tm, tn), lambda b, i, j: (b, i, j)),
        compiler_params=pltpu.CompilerParams(
            dimension_semantics=("parallel", "parallel", "parallel")),
    )


def _pad_bias(b, Np):
    N = b.shape[0]
    bp = b.astype(jnp.float32).reshape(1, N)
    if Np != N:
        bp = jnp.pad(bp, ((0, 0), (0, Np - N)))
    return bp


def _conv3x3_s1(x, w, b, relu=True):
    B, H, W, C = x.shape
    N = b.shape[0]
    Np = w.shape[1]
    wp = _ru(W + 2, 16)
    M_img = H * wp
    tm = _pick_tm(M_img)
    L = _ru(M_img + 2 * wp + 16, 16)
    xp = jnp.pad(x, ((0, 0), (1, 1), (1, wp - W - 1), (0, 0)))
    xp = xp.reshape(B, (H + 2) * wp, C)
    xp = jnp.pad(xp, ((0, 0), (0, L - (H + 2) * wp), (0, 0)))
    offs = tuple((di * wp, dj) for di in range(3) for dj in range(3))
    fn = _taps_call(B, L, C, offs, Np, M_img, tm, Np, bool(relu))
    out = fn(xp, w[:9 * C], _pad_bias(b, Np))
    return out[:, :, :N].reshape(B, H, wp, N)[:, :, :W, :]


def _conv3x3_s2(x, w, b, relu=True):
    B, H, W, C = x.shape
    H2, W2 = H // 2, W // 2
    N = b.shape[0]
    Np = w.shape[1]
    x2 = x.reshape(B, H2, 2, W2, 2, C).transpose(0, 1, 3, 2, 4, 5)
    x2 = x2.reshape(B, H2, W2, 4 * C)
    w9 = w[:9 * C].reshape(3, 3, C, Np)
    w16 = jnp.zeros((4, 4, C, Np), w.dtype).at[1:, 1:].set(w9)
    wk = w16.reshape(2, 2, 2, 2, C, Np).transpose(0, 2, 1, 3, 4, 5)
    wk = wk.reshape(16 * C, Np)
    wp = _ru(W2 + 2, 16)
    M_img = H2 * wp
    tm = _pick_tm(M_img)
    L = (H2 + 2) * wp
    xp = jnp.pad(x2, ((0, 0), (1, 1), (1, wp - W2 - 1), (0, 0)))
    xp = xp.reshape(B, L, 4 * C)
    offs = ((0, 0), (0, 1), (wp, 0), (wp, 1))
    fn = _taps_call(B, L, 4 * C, offs, Np, M_img, tm, Np, bool(relu))
    out = fn(xp, wk, _pad_bias(b, Np))
    return out[:, :, :N].reshape(B, H2, wp, N)[:, :, :W2, :]


def _conv1x1(x, w, b, relu=False, residual=None, stride=1):
    B, H, W, C = x.shape
    if stride != 1:
        x = x[:, ::stride, ::stride, :]
        H, W = x.shape[1], x.shape[2]
    a = x.reshape(B * H * W, C)
    out = _gemm(a, w, b, relu=relu, residual=residual)
    return out, (B, H, W)


def _stem(x_nchw, w, b):
    B = x_nchw.shape[0]
    N = b.shape[0]
    Np = w.shape[1]
    x = x_nchw.astype(jnp.bfloat16).reshape(B, 3, 112, 2, 112, 2)
    x2 = x.transpose(0, 2, 4, 1, 3, 5).reshape(B, 112, 112, 12)
    wp = 128
    xp = jnp.pad(x2, ((0, 0), (2, 1), (2, wp - 112 - 2), (0, 4)))
    M_img = 112 * wp
    L = _ru(115 * wp + 16, 16)
    xp = xp.reshape(B, 115 * wp, 16)
    xp = jnp.pad(xp, ((0, 0), (0, L - 115 * wp), (0, 0)))
    w7 = w[:147].reshape(7, 7, 3, Np)
    w8 = jnp.zeros((8, 8, 3, Np), w.dtype).at[1:, 1:].set(w7)
    wk = w8.reshape(4, 2, 4, 2, 3, Np).transpose(0, 2, 4, 1, 3, 5)
    wk = jnp.pad(wk.reshape(4, 4, 12, Np), ((0, 0), (0, 0), (0, 4), (0, 0)))
    wk = wk.reshape(256, Np)
    offs = tuple((I * wp, J) for I in range(4) for J in range(4))
    fn = _taps_call(B, L, 16, offs, Np, M_img, 256, Np, True)
    out = fn(xp, wk, _pad_bias(b, Np))
    return out[:, :, :N].reshape(B, 112, wp, N)[:, :, :112, :]


def _maxpool_3x3_s2(x):
    B, H, W, C = x.shape
    xp = jnp.pad(x, ((0, 0), (1, 1), (1, 1), (0, 0)),
                 constant_values=-jnp.inf)
    Ho = (H - 1) // 2 + 1
    Wo = (W - 1) // 2 + 1
    out = None
    for i in range(3):
        for j in range(3):
            s = xp[:, i:i + 2 * Ho - 1:2, j:j + 2 * Wo - 1:2, :]
            out = s if out is None else jnp.maximum(out, s)
    return out


def _head_body(a_ref, w_ref, b_ref, o_ref):
    hw = a_ref.shape[1]
    xm = jnp.sum(a_ref[...].astype(jnp.float32), axis=1) * (1.0 / hw)
    o_ref[...] = jnp.dot(xm.astype(jnp.bfloat16), w_ref[...],
                         preferred_element_type=jnp.float32) + b_ref[...]


@functools.lru_cache(maxsize=None)
def _head_call(B, HW, K, Np, tn):
    grid = (Np // tn,)
    return pl.pallas_call(
        _head_body,
        out_shape=jax.ShapeDtypeStruct((B, Np), jnp.float32),
        grid=grid,
        in_specs=[
            pl.BlockSpec((B, HW, K), lambda j: (0, 0, 0)),
            pl.BlockSpec((K, tn), lambda j: (0, j)),
            pl.BlockSpec((1, tn), lambda j: (0, j)),
        ],
        out_specs=pl.BlockSpec((B, tn), lambda j: (0, j)),
        compiler_params=pltpu.CompilerParams(
            dimension_semantics=("parallel",)),
    )


def _head(x, w, b):
    B, H, W, C = x.shape
    K, Np = w.shape
    assert K == C
    N = b.shape[0]
    xr = x.reshape(B, H * W, C)
    fn = _head_call(B, H * W, C, Np, min(Np, 512))
    return fn(xr, w, _pad_bias(b, Np))[:, :N]


def _bottleneck(x, blk, stride):
    c1w, c1b, c2w, c2b, c3w, c3b = blk[:6]
    out, _ = _conv1x1(x, c1w, c1b, relu=True)
    B, H, W, _ = x.shape
    out = out[:, :c1b.shape[0]].reshape(B, H, W, c1b.shape[0])
    if stride == 1:
        out = _conv3x3_s1(out, c2w, c2b, relu=True)
    else:
        out = _conv3x3_s2(out, c2w, c2b, relu=True)
    if len(blk) > 6:
        idn, _ = _conv1x1(x, blk[6], blk[7], relu=False, stride=stride)
    else:
        Ho, Wo = out.shape[1], out.shape[2]
        idn = x.reshape(B * Ho * Wo, -1)
    Bo, Ho, Wo, Co = out.shape
    res, _ = _conv1x1(out, c3w, c3b, relu=True, residual=idn)
    N3 = c3b.shape[0]
    return res[:, :N3].reshape(Bo, Ho, Wo, N3)


def _forward(x_nchw, stem_w, stem_b, blocks, fc_w, fc_b):
    x = _stem(x_nchw, stem_w, stem_b)
    x = _maxpool_3x3_s2(x)
    cfg = [3, 4, 6, 3]
    bi = 0
    for si, nb in enumerate(cfg):
        for k in range(nb):
            stride = 2 if (si > 0 and k == 0) else 1
            x = _bottleneck(x, blocks[bi], stride)
            bi += 1
    return _head(x, fc_w, fc_b)


_forward_jit = jax.jit(_forward, static_argnames=())


def kernel(x_nchw, stem_w, stem_b, s0_b0_conv1_w, s0_b0_conv1_b, s0_b0_conv2_w, s0_b0_conv2_b, s0_b0_conv3_w, s0_b0_conv3_b, s0_b0_downsample_w, s0_b0_downsample_b, s0_b1_conv1_w, s0_b1_conv1_b, s0_b1_conv2_w, s0_b1_conv2_b, s0_b1_conv3_w, s0_b1_conv3_b, s0_b2_conv1_w, s0_b2_conv1_b, s0_b2_conv2_w, s0_b2_conv2_b, s0_b2_conv3_w, s0_b2_conv3_b, s1_b0_conv1_w, s1_b0_conv1_b, s1_b0_conv2_w, s1_b0_conv2_b, s1_b0_conv3_w, s1_b0_conv3_b, s1_b0_downsample_w, s1_b0_downsample_b, s1_b1_conv1_w, s1_b1_conv1_b, s1_b1_conv2_w, s1_b1_conv2_b, s1_b1_conv3_w, s1_b1_conv3_b, s1_b2_conv1_w, s1_b2_conv1_b, s1_b2_conv2_w, s1_b2_conv2_b, s1_b2_conv3_w, s1_b2_conv3_b, s1_b3_conv1_w, s1_b3_conv1_b, s1_b3_conv2_w, s1_b3_conv2_b, s1_b3_conv3_w, s1_b3_conv3_b, s2_b0_conv1_w, s2_b0_conv1_b, s2_b0_conv2_w, s2_b0_conv2_b, s2_b0_conv3_w, s2_b0_conv3_b, s2_b0_downsample_w, s2_b0_downsample_b, s2_b1_conv1_w, s2_b1_conv1_b, s2_b1_conv2_w, s2_b1_conv2_b, s2_b1_conv3_w, s2_b1_conv3_b, s2_b2_conv1_w, s2_b2_conv1_b, s2_b2_conv2_w, s2_b2_conv2_b, s2_b2_conv3_w, s2_b2_conv3_b, s2_b3_conv1_w, s2_b3_conv1_b, s2_b3_conv2_w, s2_b3_conv2_b, s2_b3_conv3_w, s2_b3_conv3_b, s2_b4_conv1_w, s2_b4_conv1_b, s2_b4_conv2_w, s2_b4_conv2_b, s2_b4_conv3_w, s2_b4_conv3_b, s2_b5_conv1_w, s2_b5_conv1_b, s2_b5_conv2_w, s2_b5_conv2_b, s2_b5_conv3_w, s2_b5_conv3_b, s3_b0_conv1_w, s3_b0_conv1_b, s3_b0_conv2_w, s3_b0_conv2_b, s3_b0_conv3_w, s3_b0_conv3_b, s3_b0_downsample_w, s3_b0_downsample_b, s3_b1_conv1_w, s3_b1_conv1_b, s3_b1_conv2_w, s3_b1_conv2_b, s3_b1_conv3_w, s3_b1_conv3_b, s3_b2_conv1_w, s3_b2_conv1_b, s3_b2_conv2_w, s3_b2_conv2_b, s3_b2_conv3_w, s3_b2_conv3_b, fc_w, fc_b):
    A = dict(locals())
    cfg = [3, 4, 6, 3]
    blocks = []
    for si, nb in enumerate(cfg):
        for b in range(nb):
            names = [f's{si}_b{b}_{n}_{t}'
                     for n in ('conv1', 'conv2', 'conv3') for t in ('w', 'b')]
            blk = [A[n] for n in names]
            dwn = f's{si}_b{b}_downsample_w'
            if dwn in A:
                blk += [A[dwn], A[f's{si}_b{b}_downsample_b']]
            blocks.append(tuple(blk))
    return _forward_jit(x_nchw, stem_w, stem_b, blocks, fc_w, fc_b)


# speedup vs baseline: 3.3479x; 1.9005x over previous
import functools

import jax
import jax.numpy as jnp
from jax.experimental import pallas as pl
from jax.experimental.pallas import tpu as pltpu


def _ru(x, m):
    return ((x + m - 1) // m) * m


def _pick_tm(m):
    for t in (256, 224, 192, 160, 128, 112, 96, 80, 64, 48, 32, 16):
        if m % t == 0:
            return t
    return 0


def _gemm_body(relu, has_res):
    if has_res:
        def body(a_ref, w_ref, b_ref, r_ref, o_ref):
            y = jnp.dot(a_ref[...], w_ref[...],
                        preferred_element_type=jnp.float32)
            y = y + b_ref[...] + r_ref[...].astype(jnp.float32)
            if relu:
                y = jnp.maximum(y, 0.0)
            o_ref[...] = y.astype(o_ref.dtype)
    else:
        def body(a_ref, w_ref, b_ref, o_ref):
            y = jnp.dot(a_ref[...], w_ref[...],
                        preferred_element_type=jnp.float32)
            y = y + b_ref[...]
            if relu:
                y = jnp.maximum(y, 0.0)
            o_ref[...] = y.astype(o_ref.dtype)
    return body


@functools.lru_cache(maxsize=None)
def _gemm_call(Mp, K, Np, tm, tn, relu, has_res, out_dtype):
    grid = (Mp // tm, Np // tn)
    in_specs = [
        pl.BlockSpec((tm, K), lambda i, j: (i, 0)),
        pl.BlockSpec((K, tn), lambda i, j: (0, j)),
        pl.BlockSpec((1, tn), lambda i, j: (0, j)),
    ]
    if has_res:
        in_specs.append(pl.BlockSpec((tm, tn), lambda i, j: (i, j)))
    return pl.pallas_call(
        _gemm_body(relu, has_res),
        out_shape=jax.ShapeDtypeStruct((Mp, Np), out_dtype),
        grid=grid,
        in_specs=in_specs,
        out_specs=pl.BlockSpec((tm, tn), lambda i, j: (i, j)),
        compiler_params=pltpu.CompilerParams(
            dimension_semantics=("parallel", "parallel")),
    )


def _gemm(a, w, b, relu=False, residual=None, out_dtype=jnp.bfloat16):
    M, K = a.shape
    Kw, Np = w.shape
    assert Kw == K, (Kw, K)
    N = b.shape[0]
    tm = _pick_tm(M)
    Mp = M
    if tm == 0:
        tm = 256
        Mp = _ru(M, 256)
        a = jnp.pad(a, ((0, Mp - M), (0, 0)))
    tn = min(Np, 256)
    bp = b.astype(jnp.float32).reshape(1, N)
    if Np != N:
        bp = jnp.pad(bp, ((0, 0), (0, Np - N)))
    args = [a, w, bp]
    if residual is not None:
        r = residual
        assert r.shape[1] == Np
        if Mp != M:
            r = jnp.pad(r, ((0, Mp - M), (0, 0)))
        args.append(r)
    fn = _gemm_call(Mp, K, Np, tm, tn, bool(relu), residual is not None,
                    out_dtype)
    return fn(*args)[:M]


def _taps_body(offs, cin, tm, relu, mask=None):
    def body(a_ref, w_ref, b_ref, o_ref):
        i = pl.program_id(1)
        m0 = i * tm
        acc = None
        for t, (off_row, off_col) in enumerate(offs):
            base = pl.multiple_of(m0 + off_row, 16)
            raw = a_ref[0, pl.ds(base, tm + 16), :]
            a = raw[off_col:off_col + tm, :]
            p = jnp.dot(a, w_ref[t * cin:(t + 1) * cin, :],
                        preferred_element_type=jnp.float32)
            acc = p if acc is None else acc + p
        y = acc + b_ref[...]
        if relu:
            y = jnp.maximum(y, 0.0)
        if mask is not None:
            wp_m, w_valid, band_lo, band_hi = mask
            r = m0 + jax.lax.broadcasted_iota(jnp.int32, (tm, 1), 0)
            ok = (r >= band_lo) & (r < band_hi) & ((r % wp_m) < w_valid)
            y = jnp.where(ok, y, 0.0)
        o_ref[0] = y.astype(o_ref.dtype)
    return body


@functools.lru_cache(maxsize=None)
def _taps_call(B, L, cin, offs, Np, Mp, tm, tn, relu, mask=None):
    Kw = len(offs) * cin
    grid = (B, Mp // tm, Np // tn)
    return pl.pallas_call(
        _taps_body(offs, cin, tm, relu, mask),
        out_shape=jax.ShapeDtypeStruct((B, Mp, Np), jnp.bfloat16),
        grid=grid,
        in_specs=[
            pl.BlockSpec((1, L, cin), lambda b, i, j: (b, 0, 0)),
            pl.BlockSpec((Kw, tn), lambda b, i, j: (0, j)),
            pl.BlockSpec((1, tn), lambda b, i, j: (0, j)),
        ],
        out_specs=pl.BlockSpec((1, tm, tn), lambda b, i, j: (b, i, j)),
        compiler_params=pltpu.CompilerParams(
            dimension_semantics=("parallel", "parallel", "parallel")),
    )


def _pad_bias(b, Np):
    N = b.shape[0]
    bp = b.astype(jnp.float32).reshape(1, N)
    if Np != N:
        bp = jnp.pad(bp, ((0, 0), (0, Np - N)))
    return bp


def _conv3x3_s1(x, w, b, relu=True):
    B, H, W, C = x.shape
    N = b.shape[0]
    Np = w.shape[1]
    wp = _ru(W + 2, 16)
    M_img = H * wp
    tm = _pick_tm(M_img)
    L = _ru(M_img + 2 * wp + 16, 16)
    xp = jnp.pad(x, ((0, 0), (1, 1), (1, wp - W - 1), (0, 0)))
    xp = xp.reshape(B, (H + 2) * wp, C)
    xp = jnp.pad(xp, ((0, 0), (0, L - (H + 2) * wp), (0, 0)))
    offs = tuple((di * wp, dj) for di in range(3) for dj in range(3))
    fn = _taps_call(B, L, C, offs, Np, M_img, tm, Np, bool(relu))
    out = fn(xp, w[:9 * C], _pad_bias(b, Np))
    return out[:, :, :N].reshape(B, H, wp, N)[:, :, :W, :]


def _conv3x3_s2(x, w, b, relu=True):
    B, H, W, C = x.shape
    H2, W2 = H // 2, W // 2
    N = b.shape[0]
    Np = w.shape[1]
    x2 = x.reshape(B, H2, 2, W2, 2, C).transpose(0, 1, 3, 2, 4, 5)
    x2 = x2.reshape(B, H2, W2, 4 * C)
    w9 = w[:9 * C].reshape(3, 3, C, Np)
    w16 = jnp.zeros((4, 4, C, Np), w.dtype).at[1:, 1:].set(w9)
    wk = w16.reshape(2, 2, 2, 2, C, Np).transpose(0, 2, 1, 3, 4, 5)
    wk = wk.reshape(16 * C, Np)
    wp = _ru(W2 + 2, 16)
    M_img = H2 * wp
    tm = _pick_tm(M_img)
    L = (H2 + 2) * wp
    xp = jnp.pad(x2, ((0, 0), (1, 1), (1, wp - W2 - 1), (0, 0)))
    xp = xp.reshape(B, L, 4 * C)
    offs = ((0, 0), (0, 1), (wp, 0), (wp, 1))
    fn = _taps_call(B, L, 4 * C, offs, Np, M_img, tm, Np, bool(relu))
    out = fn(xp, wk, _pad_bias(b, Np))
    return out[:, :, :N].reshape(B, H2, wp, N)[:, :, :W2, :]


def _conv1x1(x, w, b, relu=False, residual=None, stride=1):
    B, H, W, C = x.shape
    if stride != 1:
        x = x[:, ::stride, ::stride, :]
        H, W = x.shape[1], x.shape[2]
    a = x.reshape(B * H * W, C)
    out = _gemm(a, w, b, relu=relu, residual=residual)
    return out, (B, H, W)


def _stem(x_nchw, w, b):
    B = x_nchw.shape[0]
    N = b.shape[0]
    Np = w.shape[1]
    x = x_nchw.astype(jnp.bfloat16).reshape(B, 3, 112, 2, 112, 2)
    x2 = x.transpose(0, 2, 4, 1, 3, 5).reshape(B, 112, 112, 12)
    wp = 128
    xp = jnp.pad(x2, ((0, 0), (4, 3), (2, wp - 112 - 2), (0, 4)))
    M_img = 116 * wp
    L = _ru(M_img - 256 + 3 * wp + 272, 16)
    xp = xp.reshape(B, 119 * wp, 16)
    xp = jnp.pad(xp, ((0, 0), (0, L - 119 * wp), (0, 0)))
    w7 = w[:147].reshape(7, 7, 3, Np)
    w8 = jnp.zeros((8, 8, 3, Np), w.dtype).at[1:, 1:].set(w7)
    wk = w8.reshape(4, 2, 4, 2, 3, Np).transpose(0, 2, 4, 1, 3, 5)
    wk = jnp.pad(wk.reshape(4, 4, 12, Np), ((0, 0), (0, 0), (0, 4), (0, 0)))
    wk = wk.reshape(256, Np)
    offs = tuple((I * wp, J) for I in range(4) for J in range(4))
    fn = _taps_call(B, L, 16, offs, Np, M_img, 256, Np, True,
                    mask=(wp, 112, 256, 14592))
    return fn(xp, wk, _pad_bias(b, Np))


def _pool_body(a_ref, o_ref):
    q = pl.program_id(1)
    m = None
    for dy in range(3):
        base = pl.multiple_of(q * 3584 + dy * 128 + 112, 16)
        raw = a_ref[0, pl.ds(base, 3616), :]
        for dx in range(3):
            s = raw[15 + dx:15 + dx + 3584, :]
            m = s if m is None else jnp.maximum(m, s)
    m = m.reshape(14, 2, 128, 128)[:, 0]
    m = m.reshape(896, 2, 128)[:, 0, :]
    o_ref[0] = m


@functools.lru_cache(maxsize=None)
def _pool_call(B):
    return pl.pallas_call(
        _pool_body,
        out_shape=jax.ShapeDtypeStruct((B, 3584, 128), jnp.bfloat16),
        grid=(B, 4),
        in_specs=[pl.BlockSpec((1, 14848, 128), lambda b, q: (b, 0, 0))],
        out_specs=pl.BlockSpec((1, 896, 128), lambda b, q: (b, q, 0)),
        compiler_params=pltpu.CompilerParams(
            dimension_semantics=("parallel", "parallel")),
    )


def _maxpool_3x3_s2(x_flat):
    B = x_flat.shape[0]
    out = _pool_call(B)(x_flat)
    return out.reshape(B, 56, 64, 128)[:, :, :56, :64]


def _head_body(a_ref, w_ref, b_ref, o_ref):
    hw = a_ref.shape[1]
    xm = jnp.sum(a_ref[...].astype(jnp.float32), axis=1) * (1.0 / hw)
    o_ref[...] = jnp.dot(xm.astype(jnp.bfloat16), w_ref[...],
                         preferred_element_type=jnp.float32) + b_ref[...]


@functools.lru_cache(maxsize=None)
def _head_call(B, HW, K, Np, tn):
    grid = (Np // tn,)
    return pl.pallas_call(
        _head_body,
        out_shape=jax.ShapeDtypeStruct((B, Np), jnp.float32),
        grid=grid,
        in_specs=[
            pl.BlockSpec((B, HW, K), lambda j: (0, 0, 0)),
            pl.BlockSpec((K, tn), lambda j: (0, j)),
            pl.BlockSpec((1, tn), lambda j: (0, j)),
        ],
        out_specs=pl.BlockSpec((B, tn), lambda j: (0, j)),
        compiler_params=pltpu.CompilerParams(
            dimension_semantics=("parallel",)),
    )


def _head(x, w, b):
    B, H, W, C = x.shape
    K, Np = w.shape
    assert K == C
    N = b.shape[0]
    xr = x.reshape(B, H * W, C)
    fn = _head_call(B, H * W, C, Np, min(Np, 512))
    return fn(xr, w, _pad_bias(b, Np))[:, :N]


def _bottleneck(x, blk, stride):
    c1w, c1b, c2w, c2b, c3w, c3b = blk[:6]
    out, _ = _conv1x1(x, c1w, c1b, relu=True)
    B, H, W, _ = x.shape
    out = out[:, :c1b.shape[0]].reshape(B, H, W, c1b.shape[0])
    if stride == 1:
        out = _conv3x3_s1(out, c2w, c2b, relu=True)
    else:
        out = _conv3x3_s2(out, c2w, c2b, relu=True)
    if len(blk) > 6:
        idn, _ = _conv1x1(x, blk[6], blk[7], relu=False, stride=stride)
    else:
        Ho, Wo = out.shape[1], out.shape[2]
        idn = x.reshape(B * Ho * Wo, -1)
    Bo, Ho, Wo, Co = out.shape
    res, _ = _conv1x1(out, c3w, c3b, relu=True, residual=idn)
    N3 = c3b.shape[0]
    return res[:, :N3].reshape(Bo, Ho, Wo, N3)


def _forward(x_nchw, stem_w, stem_b, blocks, fc_w, fc_b):
    x = _stem(x_nchw, stem_w, stem_b)
    x = _maxpool_3x3_s2(x)
    cfg = [3, 4, 6, 3]
    bi = 0
    for si, nb in enumerate(cfg):
        for k in range(nb):
            stride = 2 if (si > 0 and k == 0) else 1
            x = _bottleneck(x, blocks[bi], stride)
            bi += 1
    return _head(x, fc_w, fc_b)


_forward_jit = jax.jit(_forward, static_argnames=())


def kernel(x_nchw, stem_w, stem_b, s0_b0_conv1_w, s0_b0_conv1_b, s0_b0_conv2_w, s0_b0_conv2_b, s0_b0_conv3_w, s0_b0_conv3_b, s0_b0_downsample_w, s0_b0_downsample_b, s0_b1_conv1_w, s0_b1_conv1_b, s0_b1_conv2_w, s0_b1_conv2_b, s0_b1_conv3_w, s0_b1_conv3_b, s0_b2_conv1_w, s0_b2_conv1_b, s0_b2_conv2_w, s0_b2_conv2_b, s0_b2_conv3_w, s0_b2_conv3_b, s1_b0_conv1_w, s1_b0_conv1_b, s1_b0_conv2_w, s1_b0_conv2_b, s1_b0_conv3_w, s1_b0_conv3_b, s1_b0_downsample_w, s1_b0_downsample_b, s1_b1_conv1_w, s1_b1_conv1_b, s1_b1_conv2_w, s1_b1_conv2_b, s1_b1_conv3_w, s1_b1_conv3_b, s1_b2_conv1_w, s1_b2_conv1_b, s1_b2_conv2_w, s1_b2_conv2_b, s1_b2_conv3_w, s1_b2_conv3_b, s1_b3_conv1_w, s1_b3_conv1_b, s1_b3_conv2_w, s1_b3_conv2_b, s1_b3_conv3_w, s1_b3_conv3_b, s2_b0_conv1_w, s2_b0_conv1_b, s2_b0_conv2_w, s2_b0_conv2_b, s2_b0_conv3_w, s2_b0_conv3_b, s2_b0_downsample_w, s2_b0_downsample_b, s2_b1_conv1_w, s2_b1_conv1_b, s2_b1_conv2_w, s2_b1_conv2_b, s2_b1_conv3_w, s2_b1_conv3_b, s2_b2_conv1_w, s2_b2_conv1_b, s2_b2_conv2_w, s2_b2_conv2_b, s2_b2_conv3_w, s2_b2_conv3_b, s2_b3_conv1_w, s2_b3_conv1_b, s2_b3_conv2_w, s2_b3_conv2_b, s2_b3_conv3_w, s2_b3_conv3_b, s2_b4_conv1_w, s2_b4_conv1_b, s2_b4_conv2_w, s2_b4_conv2_b, s2_b4_conv3_w, s2_b4_conv3_b, s2_b5_conv1_w, s2_b5_conv1_b, s2_b5_conv2_w, s2_b5_conv2_b, s2_b5_conv3_w, s2_b5_conv3_b, s3_b0_conv1_w, s3_b0_conv1_b, s3_b0_conv2_w, s3_b0_conv2_b, s3_b0_conv3_w, s3_b0_conv3_b, s3_b0_downsample_w, s3_b0_downsample_b, s3_b1_conv1_w, s3_b1_conv1_b, s3_b1_conv2_w, s3_b1_conv2_b, s3_b1_conv3_w, s3_b1_conv3_b, s3_b2_conv1_w, s3_b2_conv1_b, s3_b2_conv2_w, s3_b2_conv2_b, s3_b2_conv3_w, s3_b2_conv3_b, fc_w, fc_b):
    A = dict(locals())
    cfg = [3, 4, 6, 3]
    blocks = []
    for si, nb in enumerate(cfg):
        for b in range(nb):
            names = [f's{si}_b{b}_{n}_{t}'
                     for n in ('conv1', 'conv2', 'conv3') for t in ('w', 'b')]
            blk = [A[n] for n in names]
            dwn = f's{si}_b{b}_downsample_w'
            if dwn in A:
                blk += [A[dwn], A[f's{si}_b{b}_downsample_b']]
            blocks.append(tuple(blk))
    return _forward_jit(x_nchw, stem_w, stem_b, blocks, fc_w, fc_b)
